# baseline (speedup 1.0000x reference)
import jax
import jax.numpy as jnp
from jax import lax
from jax.experimental import pallas as pl
from jax.experimental.pallas import tpu as pltpu
from jax.experimental.pallas import tpu_sc as plsc

_NUM_CLASSES = 20
_IGNORE = 255
_EOS = 0.1
_W_CE, _W_DICE, _W_MASK = 2.0, 5.0, 5.0
_B, _Q, _N, _NI, _P = 2, 100, 65536, 20, 4096

_NC, _NS, _L = 2, 16, 16
_NW = _NC * _NS
_PTS = _B * _P
_TASKS = _B * _NI
_ALL = 2 * _TASKS


def _sc_body(pred_hbm, tm_hbm, idx_hbm, out_lg, out_tv,
             idxb_l, trow_l, outb_l):
    c = lax.axis_index("c")
    s = lax.axis_index("s")
    w = s * _NC + c

    def run(t):
        tt = t % _TASKS
        bb = tt // _NI
        pltpu.sync_copy(idx_hbm.at[pl.ds(bb * _P, _P)], idxb_l)

        @pl.when(t < _TASKS)
        def _():
            pltpu.sync_copy(pred_hbm.at[pl.ds(tt * _N, _N)], trow_l)

        @pl.when(t >= _TASKS)
        def _():
            pltpu.sync_copy(tm_hbm.at[pl.ds(tt * _N, _N)], trow_l)

        def g(j, carry):
            iv = jnp.clip(idxb_l[pl.ds(j * _L, _L)], 0, _N - 1)
            outb_l[pl.ds(j * _L, _L)] = plsc.load_gather(trow_l, [iv])
            return carry

        lax.fori_loop(0, _P // _L, g, 0)

        @pl.when(t < _TASKS)
        def _():
            pltpu.sync_copy(outb_l, out_lg.at[pl.ds(tt * _P, _P)])

        @pl.when(t >= _TASKS)
        def _():
            pltpu.sync_copy(outb_l, out_tv.at[pl.ds(tt * _P, _P)])

    run(w)
    run(w + _NW)

    @pl.when(w < _ALL - 2 * _NW)
    def _():
        run(w + 2 * _NW)


def _sc_gather(pred_t1d, tm_1d, idx_flat):
    mesh = plsc.VectorSubcoreMesh(core_axis_name="c", subcore_axis_name="s",
                                  num_cores=_NC, num_subcores=_NS)
    f32 = jnp.float32
    return pl.kernel(
        _sc_body,
        out_type=(jax.ShapeDtypeStruct((_TASKS * _P,), f32),
                  jax.ShapeDtypeStruct((_TASKS * _P,), f32)),
        mesh=mesh,
        compiler_params=pltpu.CompilerParams(needs_layout_passes=False),
        scratch_types=[
            pltpu.VMEM((_P,), jnp.int32),
            pltpu.VMEM((_N,), f32),
            pltpu.VMEM((_P,), f32),
        ],
    )(pred_t1d, tm_1d, idx_flat)


def _tc_loss_body(x_ref, tv_ref, lg_ref, ftc_ref, out_ref):
    f32 = jnp.float32
    x = x_ref[...]
    y = (tv_ref[...] > 0.5).astype(f32)
    nm = float(_B * _NI)

    bce = jnp.maximum(x, 0.0) - x * y + jnp.log1p(jnp.exp(-jnp.abs(x)))
    loss_mask = jnp.sum(bce) / (float(_P) * nm)

    sg = 1.0 / (1.0 + jnp.exp(-x))
    num = 2.0 * jnp.sum(sg * y, axis=1)
    den = jnp.sum(sg, axis=1) + jnp.sum(y, axis=1)
    loss_dice = jnp.sum(1.0 - (num + 1.0) / (den + 1.0)) / nm

    lg = jnp.clip(lg_ref[...], -100.0, 100.0)
    m = jnp.max(lg, axis=-1, keepdims=True)
    lse = m + jnp.log(jnp.sum(jnp.exp(lg - m), axis=-1, keepdims=True))
    logp = lg - lse
    ftc = ftc_ref[...]
    cio = lax.broadcasted_iota(jnp.int32, (_B * _Q, _NUM_CLASSES + 1), 1)
    nll = -jnp.sum(jnp.where(cio == ftc, logp, 0.0), axis=-1, keepdims=True)
    wgt = jnp.where(ftc == 0, 0.0,
                    jnp.where(ftc == _NUM_CLASSES, _EOS, 1.0))
    wv = wgt * (ftc != _IGNORE).astype(f32)
    loss_ce = jnp.sum(wv * nll) / jnp.maximum(jnp.sum(wv), 1e-8)

    li = lax.broadcasted_iota(jnp.int32, (8, 128), 1)
    out_ref[...] = jnp.where(
        li == 0, loss_ce * _W_CE,
        jnp.where(li == 1, loss_dice * _W_DICE,
                  jnp.where(li == 2, loss_mask * _W_MASK, 0.0)))


def kernel(pred_logits, pred_masks, target_classes, target_masks, mask_indices):
    f32 = jnp.float32
    pred_t1d = jnp.transpose(pred_masks[:, :, :_NI],
                             (0, 2, 1)).reshape(_TASKS * _N)
    tm_1d = target_masks.reshape(_TASKS * _N)
    idx_flat = mask_indices.astype(jnp.int32).reshape(_PTS)

    logits_1d, tvals_1d = _sc_gather(pred_t1d, tm_1d, idx_flat)

    full_tc = jnp.full((_B, _Q), _NUM_CLASSES, jnp.int32)
    full_tc = full_tc.at[:, :_NI].set(target_classes.astype(jnp.int32))
    ftc2d = full_tc.reshape(_B * _Q, 1)
    lg2d = pred_logits.astype(f32).reshape(_B * _Q, _NUM_CLASSES + 1)

    out = pl.pallas_call(
        _tc_loss_body,
        out_shape=jax.ShapeDtypeStruct((8, 128), f32),
    )(logits_1d.reshape(_TASKS, _P), tvals_1d.reshape(_TASKS, _P),
      lg2d, ftc2d)
    return out[0, :3]


# speedup vs baseline: 1.1847x; 1.1847x over previous
import jax
import jax.numpy as jnp
from jax import lax
from jax.experimental import pallas as pl
from jax.experimental.pallas import tpu as pltpu
from jax.experimental.pallas import tpu_sc as plsc

_NUM_CLASSES = 20
_IGNORE = 255
_EOS = 0.1
_W_CE, _W_DICE, _W_MASK = 2.0, 5.0, 5.0
_B, _Q, _N, _NI, _P = 2, 100, 65536, 20, 4096

_NC, _NS, _L = 2, 16, 16
_NW = _NC * _NS
_PTS = _B * _P
_TASKS = _B * _NI
_ALL = 2 * _TASKS


def _sc_body(pred_hbm, tm_hbm, idx_hbm, out_lg, out_tv,
             idxb_l, trow_l, outb_l):
    c = lax.axis_index("c")
    s = lax.axis_index("s")
    w = s * _NC + c

    def run(t):
        tt = t % _TASKS
        bb = tt // _NI
        ii = tt % _NI
        pltpu.sync_copy(idx_hbm.at[pl.ds(bb * _P, _P)], idxb_l)

        @pl.when(t < _TASKS)
        def _():
            pltpu.sync_copy(pred_hbm.at[pl.ds(tt * _N, _N)], trow_l)

        @pl.when(t >= _TASKS)
        def _():
            pltpu.sync_copy(tm_hbm.at[bb, ii], trow_l)

        def g(j, carry):
            iv = jnp.clip(idxb_l[pl.ds(j * _L, _L)], 0, _N - 1)
            outb_l[pl.ds(j * _L, _L)] = plsc.load_gather(trow_l, [iv])
            return carry

        lax.fori_loop(0, _P // _L, g, 0)

        @pl.when(t < _TASKS)
        def _():
            pltpu.sync_copy(outb_l, out_lg.at[pl.ds(tt * _P, _P)])

        @pl.when(t >= _TASKS)
        def _():
            pltpu.sync_copy(outb_l, out_tv.at[pl.ds(tt * _P, _P)])

    run(w)
    run(w + _NW)

    @pl.when(w < _ALL - 2 * _NW)
    def _():
        run(w + 2 * _NW)


def _sc_gather(pred_t1d, tm_3d, idx_flat):
    mesh = plsc.VectorSubcoreMesh(core_axis_name="c", subcore_axis_name="s",
                                  num_cores=_NC, num_subcores=_NS)
    f32 = jnp.float32
    return pl.kernel(
        _sc_body,
        out_type=(jax.ShapeDtypeStruct((_TASKS * _P,), f32),
                  jax.ShapeDtypeStruct((_TASKS * _P,), f32)),
        mesh=mesh,
        compiler_params=pltpu.CompilerParams(needs_layout_passes=False),
        scratch_types=[
            pltpu.VMEM((_P,), jnp.int32),
            pltpu.VMEM((_N,), f32),
            pltpu.VMEM((_P,), f32),
        ],
    )(pred_t1d, tm_3d, idx_flat)


def _tc_loss_body(x_ref, tv_ref, lg_ref, ftc_ref, out_ref):
    f32 = jnp.float32
    x = x_ref[...]
    y = (tv_ref[...] > 0.5).astype(f32)
    nm = float(_B * _NI)

    bce = jnp.maximum(x, 0.0) - x * y + jnp.log1p(jnp.exp(-jnp.abs(x)))
    loss_mask = jnp.sum(bce) / (float(_P) * nm)

    sg = 1.0 / (1.0 + jnp.exp(-x))
    num = 2.0 * jnp.sum(sg * y, axis=1)
    den = jnp.sum(sg, axis=1) + jnp.sum(y, axis=1)
    loss_dice = jnp.sum(1.0 - (num + 1.0) / (den + 1.0)) / nm

    lg = jnp.clip(lg_ref[...], -100.0, 100.0)
    m = jnp.max(lg, axis=-1, keepdims=True)
    lse = m + jnp.log(jnp.sum(jnp.exp(lg - m), axis=-1, keepdims=True))
    logp = lg - lse
    ftc = ftc_ref[...]
    cio = lax.broadcasted_iota(jnp.int32, (_B * _Q, _NUM_CLASSES + 1), 1)
    nll = -jnp.sum(jnp.where(cio == ftc, logp, 0.0), axis=-1, keepdims=True)
    wgt = jnp.where(ftc == 0, 0.0,
                    jnp.where(ftc == _NUM_CLASSES, _EOS, 1.0))
    wv = wgt * (ftc != _IGNORE).astype(f32)
    loss_ce = jnp.sum(wv * nll) / jnp.maximum(jnp.sum(wv), 1e-8)

    li = lax.broadcasted_iota(jnp.int32, (8, 128), 1)
    out_ref[...] = jnp.where(
        li == 0, loss_ce * _W_CE,
        jnp.where(li == 1, loss_dice * _W_DICE,
                  jnp.where(li == 2, loss_mask * _W_MASK, 0.0)))


def kernel(pred_logits, pred_masks, target_classes, target_masks, mask_indices):
    f32 = jnp.float32
    pred_t1d = jnp.transpose(pred_masks[:, :, :_NI],
                             (0, 2, 1)).reshape(_TASKS * _N)
    idx_flat = mask_indices.astype(jnp.int32).reshape(_PTS)

    logits_1d, tvals_1d = _sc_gather(pred_t1d, target_masks, idx_flat)

    full_tc = jnp.full((_B, _Q), _NUM_CLASSES, jnp.int32)
    full_tc = full_tc.at[:, :_NI].set(target_classes.astype(jnp.int32))
    ftc2d = full_tc.reshape(_B * _Q, 1)
    lg2d = pred_logits.astype(f32).reshape(_B * _Q, _NUM_CLASSES + 1)

    out = pl.pallas_call(
        _tc_loss_body,
        out_shape=jax.ShapeDtypeStruct((8, 128), f32),
    )(logits_1d.reshape(_TASKS, _P), tvals_1d.reshape(_TASKS, _P),
      lg2d, ftc2d)
    return out[0, :3]


# speedup vs baseline: 1.1892x; 1.0039x over previous
import jax
import jax.numpy as jnp
from jax import lax
from jax.experimental import pallas as pl
from jax.experimental.pallas import tpu as pltpu
from jax.experimental.pallas import tpu_sc as plsc

_NUM_CLASSES = 20
_IGNORE = 255
_EOS = 0.1
_W_CE, _W_DICE, _W_MASK = 2.0, 5.0, 5.0
_B, _Q, _N, _NI, _P = 2, 100, 65536, 20, 4096

_NC, _NS, _L = 2, 16, 16
_NW = _NC * _NS
_PTS = _B * _P
_TM_TASKS = _B * _NI
_PPW = _P // _NS
_NG = _NI * _PPW // 128


def _sc_body(pred_hbm, tm_hbm, idx_hbm, out_lg, out_tv,
             idxp_l, idxg_l, outp_l, idxb_l, trow_l, outb_l, sem):
    c = lax.axis_index("c")
    s = lax.axis_index("s")

    pltpu.sync_copy(idx_hbm.at[pl.ds(c * _P + s * _PPW, _PPW)], idxp_l)
    for cc in range(_B):
        @pl.when(c == cc)
        def _(cc=cc):
            for i in range(_NI):
                base = (cc * _NI + i) * _N

                def add(k, carry, i=i, base=base):
                    idxg_l[pl.ds(i * _PPW + k * _L, _L)] = (
                        idxp_l[pl.ds(k * _L, _L)] + base)
                    return carry

                lax.fori_loop(0, _PPW // _L, add, 0)

    cps = [
        pltpu.async_copy(pred_hbm.at[idxg_l.at[pl.ds(g * 128, 128)]],
                         outp_l.at[pl.ds(g * 128, 128)], sem)
        for g in range(_NG)
    ]

    w = s * _NC + c

    def run_tm(t):
        bb = t // _NI
        ii = t % _NI
        pltpu.sync_copy(idx_hbm.at[pl.ds(bb * _P, _P)], idxb_l)
        pltpu.sync_copy(tm_hbm.at[bb, ii], trow_l)

        def g(j, carry):
            iv = idxb_l[pl.ds(j * _L, _L)]
            outb_l[pl.ds(j * _L, _L)] = plsc.load_gather(trow_l, [iv])
            return carry

        lax.fori_loop(0, _P // _L, g, 0)
        pltpu.sync_copy(outb_l, out_tv.at[pl.ds(t * _P, _P)])

    run_tm(w)

    for cp in cps:
        cp.wait()
    for i in range(_NI):
        off = (c * _NI + i) * _P + s * _PPW
        pltpu.sync_copy(outp_l.at[pl.ds(i * _PPW, _PPW)],
                        out_lg.at[pl.ds(off, _PPW)])

    @pl.when(w < _TM_TASKS - _NW)
    def _():
        run_tm(w + _NW)


def _sc_gather(pred_t1d, tm_3d, idx_flat):
    mesh = plsc.VectorSubcoreMesh(core_axis_name="c", subcore_axis_name="s",
                                  num_cores=_NC, num_subcores=_NS)
    f32 = jnp.float32
    return pl.kernel(
        _sc_body,
        out_type=(jax.ShapeDtypeStruct((_TM_TASKS * _P,), f32),
                  jax.ShapeDtypeStruct((_TM_TASKS * _P,), f32)),
        mesh=mesh,
        compiler_params=pltpu.CompilerParams(needs_layout_passes=False),
        scratch_types=[
            pltpu.VMEM((_PPW,), jnp.int32),
            pltpu.VMEM((_NI * _PPW,), jnp.int32),
            pltpu.VMEM((_NI * _PPW,), f32),
            pltpu.VMEM((_P,), jnp.int32),
            pltpu.VMEM((_N,), f32),
            pltpu.VMEM((_P,), f32),
            pltpu.SemaphoreType.DMA,
        ],
    )(pred_t1d, tm_3d, idx_flat)


def _tc_loss_body(x_ref, tv_ref, lg_ref, ftc_ref, out_ref):
    f32 = jnp.float32
    x = x_ref[...]
    y = (tv_ref[...] > 0.5).astype(f32)
    nm = float(_B * _NI)

    bce = jnp.maximum(x, 0.0) - x * y + jnp.log1p(jnp.exp(-jnp.abs(x)))
    loss_mask = jnp.sum(bce) / (float(_P) * nm)

    sg = 1.0 / (1.0 + jnp.exp(-x))
    rows = _TM_TASKS
    num = 2.0 * jnp.sum((sg * y).reshape(rows, _P // 128, 128), axis=(1, 2))
    den = (jnp.sum(sg.reshape(rows, _P // 128, 128), axis=(1, 2))
           + jnp.sum(y.reshape(rows, _P // 128, 128), axis=(1, 2)))
    loss_dice = jnp.sum(1.0 - (num + 1.0) / (den + 1.0)) / nm

    lg = jnp.clip(lg_ref[...], -100.0, 100.0)
    m = jnp.max(lg, axis=-1, keepdims=True)
    lse = m + jnp.log(jnp.sum(jnp.exp(lg - m), axis=-1, keepdims=True))
    logp = lg - lse
    ftc = ftc_ref[...]
    cio = lax.broadcasted_iota(jnp.int32, (_B * _Q, _NUM_CLASSES + 1), 1)
    nll = -jnp.sum(jnp.where(cio == ftc, logp, 0.0), axis=-1, keepdims=True)
    wgt = jnp.where(ftc == 0, 0.0,
                    jnp.where(ftc == _NUM_CLASSES, _EOS, 1.0))
    wv = wgt * (ftc != _IGNORE).astype(f32)
    loss_ce = jnp.sum(wv * nll) / jnp.maximum(jnp.sum(wv), 1e-8)

    li = lax.broadcasted_iota(jnp.int32, (8, 128), 1)
    out_ref[...] = jnp.where(
        li == 0, loss_ce * _W_CE,
        jnp.where(li == 1, loss_dice * _W_DICE,
                  jnp.where(li == 2, loss_mask * _W_MASK, 0.0)))


def kernel(pred_logits, pred_masks, target_classes, target_masks, mask_indices):
    f32 = jnp.float32
    pred_t1d = jnp.transpose(pred_masks[:, :, :_NI],
                             (0, 2, 1)).reshape(_TM_TASKS * _N)
    idx_flat = jnp.clip(mask_indices.astype(jnp.int32), 0, _N - 1).reshape(_PTS)

    logits_1d, tvals_1d = _sc_gather(pred_t1d, target_masks, idx_flat)

    full_tc = jnp.full((_B, _Q), _NUM_CLASSES, jnp.int32)
    full_tc = full_tc.at[:, :_NI].set(target_classes.astype(jnp.int32))
    ftc2d = full_tc.reshape(_B * _Q, 1)
    lg2d = pred_logits.astype(f32).reshape(_B * _Q, _NUM_CLASSES + 1)

    out = pl.pallas_call(
        _tc_loss_body,
        out_shape=jax.ShapeDtypeStruct((8, 128), f32),
    )(logits_1d.reshape(_TM_TASKS * _P // 128, 128),
      tvals_1d.reshape(_TM_TASKS * _P // 128, 128),
      lg2d, ftc2d)
    return out[0, :3]


# speedup vs baseline: 1.2345x; 1.0381x over previous
import jax
import jax.numpy as jnp
from jax import lax
from jax.experimental import pallas as pl
from jax.experimental.pallas import tpu as pltpu
from jax.experimental.pallas import tpu_sc as plsc

_NUM_CLASSES = 20
_IGNORE = 255
_EOS = 0.1
_W_CE, _W_DICE, _W_MASK = 2.0, 5.0, 5.0
_B, _Q, _N, _NI, _P = 2, 100, 65536, 20, 4096

_NC, _NS, _L = 2, 16, 16
_NW = _NC * _NS
_PTS = _B * _P
_TASKS = _B * _NI
_ALL = 2 * _TASKS


def _sc_body(pred_hbm, tm_hbm, idx_hbm, out_lg, out_tv,
             idxb_l, trow_l, outb_l):
    c = lax.axis_index("c")
    s = lax.axis_index("s")
    w = s * _NC + c

    def run(t):
        tt = t % _TASKS
        bb = tt // _NI
        ii = tt % _NI
        pltpu.sync_copy(idx_hbm.at[pl.ds(bb * _P, _P)], idxb_l)

        @pl.when(t < _TASKS)
        def _():
            pltpu.sync_copy(pred_hbm.at[pl.ds(tt * _N, _N)], trow_l)

        @pl.when(t >= _TASKS)
        def _():
            pltpu.sync_copy(tm_hbm.at[bb, ii], trow_l)

        def g(j, carry):
            for u in range(2):
                o = j * 2 * _L + u * _L
                iv = idxb_l[pl.ds(o, _L)]
                outb_l[pl.ds(o, _L)] = plsc.load_gather(trow_l, [iv])
            return carry

        lax.fori_loop(0, _P // (2 * _L), g, 0)

        @pl.when(t < _TASKS)
        def _():
            pltpu.sync_copy(outb_l, out_lg.at[pl.ds(tt * _P, _P)])

        @pl.when(t >= _TASKS)
        def _():
            pltpu.sync_copy(outb_l, out_tv.at[pl.ds(tt * _P, _P)])

    run(w)
    run(w + _NW)

    @pl.when(w < _ALL - 2 * _NW)
    def _():
        run(w + 2 * _NW)


def _sc_gather(pred_t1d, tm_3d, idx_flat):
    mesh = plsc.VectorSubcoreMesh(core_axis_name="c", subcore_axis_name="s",
                                  num_cores=_NC, num_subcores=_NS)
    f32 = jnp.float32
    return pl.kernel(
        _sc_body,
        out_type=(jax.ShapeDtypeStruct((_TASKS * _P,), f32),
                  jax.ShapeDtypeStruct((_TASKS * _P,), f32)),
        mesh=mesh,
        compiler_params=pltpu.CompilerParams(needs_layout_passes=False),
        scratch_types=[
            pltpu.VMEM((_P,), jnp.int32),
            pltpu.VMEM((_N,), f32),
            pltpu.VMEM((_P,), f32),
        ],
    )(pred_t1d, tm_3d, idx_flat)


def _tc_loss_body(x_ref, tv_ref, lg_ref, ftc_ref, out_ref):
    f32 = jnp.float32
    x = x_ref[...]
    y = (tv_ref[...] > 0.5).astype(f32)
    nm = float(_B * _NI)

    bce = jnp.maximum(x, 0.0) - x * y + jnp.log1p(jnp.exp(-jnp.abs(x)))
    loss_mask = jnp.sum(bce) / (float(_P) * nm)

    sg = 1.0 / (1.0 + jnp.exp(-x))
    rows = _TASKS
    num = 2.0 * jnp.sum((sg * y).reshape(rows, _P // 128, 128), axis=(1, 2))
    den = (jnp.sum(sg.reshape(rows, _P // 128, 128), axis=(1, 2))
           + jnp.sum(y.reshape(rows, _P // 128, 128), axis=(1, 2)))
    loss_dice = jnp.sum(1.0 - (num + 1.0) / (den + 1.0)) / nm

    lg = jnp.clip(lg_ref[...], -100.0, 100.0)
    m = jnp.max(lg, axis=-1, keepdims=True)
    lse = m + jnp.log(jnp.sum(jnp.exp(lg - m), axis=-1, keepdims=True))
    logp = lg - lse
    ftc = ftc_ref[...]
    cio = lax.broadcasted_iota(jnp.int32, (_B * _Q, _NUM_CLASSES + 1), 1)
    nll = -jnp.sum(jnp.where(cio == ftc, logp, 0.0), axis=-1, keepdims=True)
    wgt = jnp.where(ftc == 0, 0.0,
                    jnp.where(ftc == _NUM_CLASSES, _EOS, 1.0))
    wv = wgt * (ftc != _IGNORE).astype(f32)
    loss_ce = jnp.sum(wv * nll) / jnp.maximum(jnp.sum(wv), 1e-8)

    li = lax.broadcasted_iota(jnp.int32, (8, 128), 1)
    out_ref[...] = jnp.where(
        li == 0, loss_ce * _W_CE,
        jnp.where(li == 1, loss_dice * _W_DICE,
                  jnp.where(li == 2, loss_mask * _W_MASK, 0.0)))


def kernel(pred_logits, pred_masks, target_classes, target_masks, mask_indices):
    f32 = jnp.float32
    pred_t1d = jnp.transpose(pred_masks[:, :, :_NI],
                             (0, 2, 1)).reshape(_TASKS * _N)
    idx_flat = jnp.clip(mask_indices.astype(jnp.int32), 0, _N - 1).reshape(_PTS)

    logits_1d, tvals_1d = _sc_gather(pred_t1d, target_masks, idx_flat)

    full_tc = jnp.full((_B, _Q), _NUM_CLASSES, jnp.int32)
    full_tc = full_tc.at[:, :_NI].set(target_classes.astype(jnp.int32))
    ftc2d = full_tc.reshape(_B * _Q, 1)
    lg2d = pred_logits.astype(f32).reshape(_B * _Q, _NUM_CLASSES + 1)

    out = pl.pallas_call(
        _tc_loss_body,
        out_shape=jax.ShapeDtypeStruct((8, 128), f32),
    )(logits_1d.reshape(_TASKS * _P // 128, 128),
      tvals_1d.reshape(_TASKS * _P // 128, 128),
      lg2d, ftc2d)
    return out[0, :3]


# speedup vs baseline: 1.2406x; 1.0050x over previous
import jax
import jax.numpy as jnp
from jax import lax
from jax.experimental import pallas as pl
from jax.experimental.pallas import tpu as pltpu
from jax.experimental.pallas import tpu_sc as plsc

_NUM_CLASSES = 20
_IGNORE = 255
_EOS = 0.1
_W_CE, _W_DICE, _W_MASK = 2.0, 5.0, 5.0
_B, _Q, _N, _NI, _P = 2, 100, 65536, 20, 4096

_NC, _NS, _L = 2, 16, 16
_NW = _NC * _NS
_PTS = _B * _P
_TASKS = _B * _NI
_ALL = 2 * _TASKS


def _sc_body(pred_hbm, tm_hbm, idx_hbm, out_lg, out_tv,
             idxb_l, trow_l, outb_l):
    c = lax.axis_index("c")
    s = lax.axis_index("s")
    w = s * _NC + c

    def run(t):
        tt = t % _TASKS
        bb = tt // _NI
        ii = tt % _NI
        pltpu.sync_copy(idx_hbm.at[pl.ds(bb * _P, _P)], idxb_l)

        @pl.when(t < _TASKS)
        def _():
            pltpu.sync_copy(pred_hbm.at[pl.ds(tt * _N, _N)], trow_l)

        @pl.when(t >= _TASKS)
        def _():
            pltpu.sync_copy(tm_hbm.at[bb, ii], trow_l)

        def g(j, carry):
            for u in range(4):
                o = j * 4 * _L + u * _L
                iv = idxb_l[pl.ds(o, _L)]
                outb_l[pl.ds(o, _L)] = plsc.load_gather(trow_l, [iv])
            return carry

        lax.fori_loop(0, _P // (4 * _L), g, 0)

        @pl.when(t < _TASKS)
        def _():
            pltpu.sync_copy(outb_l, out_lg.at[pl.ds(tt * _P, _P)])

        @pl.when(t >= _TASKS)
        def _():
            pltpu.sync_copy(outb_l, out_tv.at[pl.ds(tt * _P, _P)])

    run(w)
    run(w + _NW)

    @pl.when(w < _ALL - 2 * _NW)
    def _():
        run(w + 2 * _NW)


def _sc_gather(pred_t1d, tm_3d, idx_flat):
    mesh = plsc.VectorSubcoreMesh(core_axis_name="c", subcore_axis_name="s",
                                  num_cores=_NC, num_subcores=_NS)
    f32 = jnp.float32
    return pl.kernel(
        _sc_body,
        out_type=(jax.ShapeDtypeStruct((_TASKS * _P,), f32),
                  jax.ShapeDtypeStruct((_TASKS * _P,), f32)),
        mesh=mesh,
        compiler_params=pltpu.CompilerParams(needs_layout_passes=False),
        scratch_types=[
            pltpu.VMEM((_P,), jnp.int32),
            pltpu.VMEM((_N,), f32),
            pltpu.VMEM((_P,), f32),
        ],
    )(pred_t1d, tm_3d, idx_flat)


def _tc_loss_body(x_ref, tv_ref, lg_ref, ftc_ref, out_ref):
    f32 = jnp.float32
    x = x_ref[...]
    y = (tv_ref[...] > 0.5).astype(f32)
    nm = float(_B * _NI)

    bce = jnp.maximum(x, 0.0) - x * y + jnp.log1p(jnp.exp(-jnp.abs(x)))
    loss_mask = jnp.sum(bce) / (float(_P) * nm)

    sg = 1.0 / (1.0 + jnp.exp(-x))
    rows = _TASKS
    num = 2.0 * jnp.sum((sg * y).reshape(rows, _P // 128, 128), axis=(1, 2))
    den = (jnp.sum(sg.reshape(rows, _P // 128, 128), axis=(1, 2))
           + jnp.sum(y.reshape(rows, _P // 128, 128), axis=(1, 2)))
    loss_dice = jnp.sum(1.0 - (num + 1.0) / (den + 1.0)) / nm

    lg = jnp.clip(lg_ref[...], -100.0, 100.0)
    m = jnp.max(lg, axis=-1, keepdims=True)
    lse = m + jnp.log(jnp.sum(jnp.exp(lg - m), axis=-1, keepdims=True))
    logp = lg - lse
    ftc = ftc_ref[...]
    cio = lax.broadcasted_iota(jnp.int32, (_B * _Q, _NUM_CLASSES + 1), 1)
    nll = -jnp.sum(jnp.where(cio == ftc, logp, 0.0), axis=-1, keepdims=True)
    wgt = jnp.where(ftc == 0, 0.0,
                    jnp.where(ftc == _NUM_CLASSES, _EOS, 1.0))
    wv = wgt * (ftc != _IGNORE).astype(f32)
    loss_ce = jnp.sum(wv * nll) / jnp.maximum(jnp.sum(wv), 1e-8)

    li = lax.broadcasted_iota(jnp.int32, (8, 128), 1)
    out_ref[...] = jnp.where(
        li == 0, loss_ce * _W_CE,
        jnp.where(li == 1, loss_dice * _W_DICE,
                  jnp.where(li == 2, loss_mask * _W_MASK, 0.0)))


def kernel(pred_logits, pred_masks, target_classes, target_masks, mask_indices):
    f32 = jnp.float32
    pred_t1d = jnp.transpose(pred_masks[:, :, :_NI],
                             (0, 2, 1)).reshape(_TASKS * _N)
    idx_flat = jnp.clip(mask_indices.astype(jnp.int32), 0, _N - 1).reshape(_PTS)

    logits_1d, tvals_1d = _sc_gather(pred_t1d, target_masks, idx_flat)

    full_tc = jnp.full((_B, _Q), _NUM_CLASSES, jnp.int32)
    full_tc = full_tc.at[:, :_NI].set(target_classes.astype(jnp.int32))
    ftc2d = full_tc.reshape(_B * _Q, 1)
    lg2d = pred_logits.astype(f32).reshape(_B * _Q, _NUM_CLASSES + 1)

    out = pl.pallas_call(
        _tc_loss_body,
        out_shape=jax.ShapeDtypeStruct((8, 128), f32),
    )(logits_1d.reshape(_TASKS * _P // 128, 128),
      tvals_1d.reshape(_TASKS * _P // 128, 128),
      lg2d, ftc2d)
    return out[0, :3]
